# baseline (speedup 1.0000x reference)
import functools

import jax
import jax.numpy as jnp
from jax import lax
from jax.experimental import pallas as pl
from jax.experimental.pallas import tpu as pltpu
from jax.experimental.pallas import tpu_sc as plsc

_NC, _NS = 2, 16
_NW = _NC * _NS
_L = 16
_TM = 2048
_TB = 16


def _sc_gather(user_table_t, item_table_t, uidx, iidx):
    B = uidx.shape[0]
    D = user_table_t.shape[0]
    bpw = B // _NW
    nb = bpw // _TB
    mesh = plsc.VectorSubcoreMesh(core_axis_name="c", subcore_axis_name="s")

    @functools.partial(
        pl.kernel,
        mesh=mesh,
        compiler_params=pltpu.CompilerParams(needs_layout_passes=False,
                                             skip_device_barrier=True),
        out_type=(
            jax.ShapeDtypeStruct((D, B), jnp.float32),
            jax.ShapeDtypeStruct((D, B), jnp.float32),
        ),
        scratch_types=[
            pltpu.VMEM((bpw + _L,), jnp.int32),
            pltpu.VMEM((bpw,), jnp.int32),
            pltpu.VMEM((bpw + _L,), jnp.int32),
            pltpu.VMEM((bpw,), jnp.int32),
            pltpu.VMEM((D, _TB * 128), jnp.float32),
            pltpu.VMEM((D, _TB * 128), jnp.float32),
            pltpu.VMEM((D, bpw), jnp.float32),
            pltpu.VMEM((D, bpw), jnp.float32),
            pltpu.SemaphoreType.DMA,
            pltpu.SemaphoreType.DMA,
        ],
    )
    def gather(ut_hbm, it_hbm, ui_hbm, ii_hbm, uo_hbm, io_hbm,
               t_u, l_u, t_i, l_i, tiles_u, tiles_i, cols_u, cols_i,
               sem_u, sem_i):
        wid = lax.axis_index("s") * _NC + lax.axis_index("c")
        base = pl.multiple_of(wid * bpw, 128)
        lanes = lax.iota(jnp.int32, _L)

        for i_hbm, t_v, l_v in ((ui_hbm, t_u, l_u), (ii_hbm, t_i, l_i)):
            pltpu.sync_copy(i_hbm.at[pl.ds(base, bpw)], t_v.at[pl.ds(0, bpw)])

            def split_idx(k, _, t_v=t_v, l_v=l_v):
                v = t_v[pl.ds(k * _L, _L)]
                l_v[pl.ds(k * _L, _L)] = lax.bitwise_and(v, 127)
                t_v[pl.ds(k * _L, _L)] = lax.shift_right_logical(v, 7)
                return 0

            lax.fori_loop(0, bpw // _L, split_idx, 0)

        def params(s):
            tab, t_v, tiles, sem = (
                (ut_hbm, t_u, tiles_u, sem_u) if s % 2 == 0
                else (it_hbm, t_i, tiles_i, sem_i))
            return tab, t_v, tiles, sem, (s // 2) * _TB

        def tile_copy(tab, t_v, tiles, sem, row0, j):
            t = t_v[pl.ds(row0 + j, _L)][0]
            return pltpu.make_async_copy(
                tab.at[:, pl.ds(pl.multiple_of(t * 128, 128), 128)],
                tiles.at[:, pl.ds(pl.multiple_of(j * 128, 128), 128)],
                sem)

        def fire(s):
            tab, t_v, tiles, sem, row0 = params(s)
            lax.fori_loop(
                0, _TB,
                lambda j, _: (tile_copy(tab, t_v, tiles, sem, row0, j)
                              .start(), 0)[1],
                0)

        def drain_extract(s):
            tab, t_v, tiles, sem, row0 = params(s)
            l_v = l_u if s % 2 == 0 else l_i
            cols = cols_u if s % 2 == 0 else cols_i
            lax.fori_loop(
                0, _TB,
                lambda j, _: (tile_copy(tab, t_v, tiles, sem, row0, j)
                              .wait(), 0)[1],
                0)
            lane_vec = l_v[pl.ds(row0, _L)]
            src1 = lanes * 128 + lane_vec
            dst1 = lanes + row0
            for d in range(D):
                d_vec = jnp.full((_L,), d, jnp.int32)
                vals = plsc.load_gather(tiles, [d_vec, src1])
                plsc.store_scatter(cols, [d_vec, dst1], vals)

        fire(0)
        for s in range(2 * nb):
            if s + 1 < 2 * nb:
                fire(s + 1)
            drain_extract(s)

        pltpu.sync_copy(cols_u, uo_hbm.at[:, pl.ds(base, bpw)])
        pltpu.sync_copy(cols_i, io_hbm.at[:, pl.ds(base, bpw)])

    return gather(user_table_t, item_table_t, uidx, iidx)


def _make_loss_body(col0):
    def _loss_body(ut_ref, vt_ref, out_ref):
        i = pl.program_id(0)
        tm = ut_ref.shape[1]
        u = ut_ref[...]
        logits = lax.dot_general(u, vt_ref[...],
                                 (((0,), (0,)), ((), ())),
                                 preferred_element_type=jnp.float32)
        sp = jnp.log(1.0 + jnp.exp(logits))
        diag = jnp.sum(u * vt_ref[:, pl.ds(col0 + i * tm, tm)])
        part = jnp.sum(sp) - diag

        @pl.when(i == 0)
        def _init():
            out_ref[0, 0] = 0.0

        out_ref[0, 0] += part

    return _loss_body


def _loss_tc(u_emb_t, v_emb_t, col0):
    D, M = u_emb_t.shape
    B = v_emb_t.shape[1]
    out = pl.pallas_call(
        _make_loss_body(col0),
        grid=(M // _TM,),
        in_specs=[
            pl.BlockSpec((D, _TM), lambda i: (0, i)),
            pl.BlockSpec((D, B), lambda i: (0, 0)),
        ],
        out_specs=pl.BlockSpec(block_shape=(1, 1), index_map=lambda i: (0, 0),
                               memory_space=pltpu.SMEM),
        out_shape=jax.ShapeDtypeStruct((1, 1), jnp.float32),
        compiler_params=pltpu.CompilerParams(skip_device_barrier=True),
    )(u_emb_t, v_emb_t)
    return out[0, 0]


def kernel(userid, itemid, user_feature, item_feature, user_table, item_table):
    B = userid.shape[0]
    uidx = userid.reshape(-1)
    iidx = itemid.reshape(-1)
    u_emb_t, i_emb_t = _sc_gather(user_table.T, item_table.T, uidx, iidx)
    total = _loss_tc(u_emb_t, i_emb_t, 0)
    return total / jnp.float32(B)


# speedup vs baseline: 1.0019x; 1.0019x over previous
import functools

import jax
import jax.numpy as jnp
from jax import lax
from jax.experimental import pallas as pl
from jax.experimental.pallas import tpu as pltpu
from jax.experimental.pallas import tpu_sc as plsc

_NC, _NS = 2, 16
_NW = _NC * _NS
_L = 16
_TM = 2048
_TB = 16


def _sc_gather(user_table_t, item_table_t, uidx, iidx):
    B = uidx.shape[0]
    D = user_table_t.shape[0]
    bpw = B // _NW
    nb = bpw // _TB
    mesh = plsc.VectorSubcoreMesh(core_axis_name="c", subcore_axis_name="s")

    @functools.partial(
        pl.kernel,
        mesh=mesh,
        compiler_params=pltpu.CompilerParams(needs_layout_passes=False),
        out_type=(
            jax.ShapeDtypeStruct((D, B), jnp.float32),
            jax.ShapeDtypeStruct((D, B), jnp.float32),
        ),
        scratch_types=[
            pltpu.VMEM((bpw + _L,), jnp.int32),
            pltpu.VMEM((bpw,), jnp.int32),
            pltpu.VMEM((bpw + _L,), jnp.int32),
            pltpu.VMEM((bpw,), jnp.int32),
            pltpu.VMEM((D, _TB * 128), jnp.float32),
            pltpu.VMEM((D, _TB * 128), jnp.float32),
            pltpu.VMEM((D, bpw), jnp.float32),
            pltpu.VMEM((D, bpw), jnp.float32),
            pltpu.SemaphoreType.DMA,
            pltpu.SemaphoreType.DMA,
        ],
    )
    def gather(ut_hbm, it_hbm, ui_hbm, ii_hbm, uo_hbm, io_hbm,
               t_u, l_u, t_i, l_i, tiles_u, tiles_i, cols_u, cols_i,
               sem_u, sem_i):
        wid = lax.axis_index("s") * _NC + lax.axis_index("c")
        base = pl.multiple_of(wid * bpw, 128)
        lanes = lax.iota(jnp.int32, _L)

        for i_hbm, t_v, l_v in ((ui_hbm, t_u, l_u), (ii_hbm, t_i, l_i)):
            pltpu.sync_copy(i_hbm.at[pl.ds(base, bpw)], t_v.at[pl.ds(0, bpw)])

            def split_idx(k, _, t_v=t_v, l_v=l_v):
                v = t_v[pl.ds(k * _L, _L)]
                l_v[pl.ds(k * _L, _L)] = lax.bitwise_and(v, 127)
                t_v[pl.ds(k * _L, _L)] = lax.shift_right_logical(v, 7)
                return 0

            lax.fori_loop(0, bpw // _L, split_idx, 0)

        def params(s):
            tab, t_v, tiles, sem = (
                (ut_hbm, t_u, tiles_u, sem_u) if s % 2 == 0
                else (it_hbm, t_i, tiles_i, sem_i))
            return tab, t_v, tiles, sem, (s // 2) * _TB

        def tile_copy(tab, t_v, tiles, sem, row0, j):
            t = t_v[pl.ds(row0 + j, _L)][0]
            return pltpu.make_async_copy(
                tab.at[:, pl.ds(pl.multiple_of(t * 128, 128), 128)],
                tiles.at[:, pl.ds(pl.multiple_of(j * 128, 128), 128)],
                sem)

        def fire(s):
            tab, t_v, tiles, sem, row0 = params(s)
            lax.fori_loop(
                0, _TB,
                lambda j, _: (tile_copy(tab, t_v, tiles, sem, row0, j)
                              .start(), 0)[1],
                0)

        def drain_extract(s):
            tab, t_v, tiles, sem, row0 = params(s)
            l_v = l_u if s % 2 == 0 else l_i
            cols = cols_u if s % 2 == 0 else cols_i
            lax.fori_loop(
                0, _TB,
                lambda j, _: (tile_copy(tab, t_v, tiles, sem, row0, j)
                              .wait(), 0)[1],
                0)
            lane_vec = l_v[pl.ds(row0, _L)]
            src1 = lanes * 128 + lane_vec
            dst1 = lanes + row0
            for d in range(D):
                d_vec = jnp.full((_L,), d, jnp.int32)
                vals = plsc.load_gather(tiles, [d_vec, src1])
                plsc.store_scatter(cols, [d_vec, dst1], vals)

        fire(0)
        for s in range(2 * nb):
            if s + 1 < 2 * nb:
                fire(s + 1)
            drain_extract(s)

        pltpu.sync_copy(cols_u, uo_hbm.at[:, pl.ds(base, bpw)])
        pltpu.sync_copy(cols_i, io_hbm.at[:, pl.ds(base, bpw)])

    return gather(user_table_t, item_table_t, uidx, iidx)


_LOG2E = 1.4426950408889634
_LN2 = 0.6931471805599453


def _make_loss_body(col0):
    def _loss_body(ut_ref, vt_ref, out_ref):
        i = pl.program_id(0)
        tm = ut_ref.shape[1]
        u = ut_ref[...] * _LOG2E
        logits2 = lax.dot_general(u, vt_ref[...],
                                  (((0,), (0,)), ((), ())),
                                  preferred_element_type=jnp.float32)
        sp = lax.log(1.0 + lax.exp2(logits2))
        diag2 = jnp.sum(u * vt_ref[:, pl.ds(col0 + i * tm, tm)])
        part = jnp.sum(sp) - diag2 * _LN2

        @pl.when(i == 0)
        def _init():
            out_ref[0, 0] = 0.0

        out_ref[0, 0] += part

    return _loss_body


def _loss_tc(u_emb_t, v_emb_t, col0):
    D, M = u_emb_t.shape
    B = v_emb_t.shape[1]
    out = pl.pallas_call(
        _make_loss_body(col0),
        grid=(M // _TM,),
        in_specs=[
            pl.BlockSpec((D, _TM), lambda i: (0, i)),
            pl.BlockSpec((D, B), lambda i: (0, 0)),
        ],
        out_specs=pl.BlockSpec(block_shape=(1, 1), index_map=lambda i: (0, 0),
                               memory_space=pltpu.SMEM),
        out_shape=jax.ShapeDtypeStruct((1, 1), jnp.float32),
    )(u_emb_t, v_emb_t)
    return out[0, 0]


def kernel(userid, itemid, user_feature, item_feature, user_table, item_table):
    B = userid.shape[0]
    uidx = userid.reshape(-1)
    iidx = itemid.reshape(-1)
    u_emb_t, i_emb_t = _sc_gather(user_table.T, item_table.T, uidx, iidx)
    total = _loss_tc(u_emb_t, i_emb_t, 0)
    return total / jnp.float32(B)


# speedup vs baseline: 1.0053x; 1.0034x over previous
import functools

import jax
import jax.numpy as jnp
from jax import lax
from jax.experimental import pallas as pl
from jax.experimental.pallas import tpu as pltpu
from jax.experimental.pallas import tpu_sc as plsc

_NC, _NS = 2, 16
_NW = _NC * _NS
_L = 16
_TM = 4096
_TB = 16


def _sc_gather(user_table_t, item_table_t, uidx, iidx):
    B = uidx.shape[0]
    D = user_table_t.shape[0]
    bpw = B // _NW
    nb = bpw // _TB
    mesh = plsc.VectorSubcoreMesh(core_axis_name="c", subcore_axis_name="s")

    @functools.partial(
        pl.kernel,
        mesh=mesh,
        compiler_params=pltpu.CompilerParams(needs_layout_passes=False),
        out_type=(
            jax.ShapeDtypeStruct((D, B), jnp.float32),
            jax.ShapeDtypeStruct((D, B), jnp.float32),
        ),
        scratch_types=[
            pltpu.VMEM((bpw + _L,), jnp.int32),
            pltpu.VMEM((bpw,), jnp.int32),
            pltpu.VMEM((bpw + _L,), jnp.int32),
            pltpu.VMEM((bpw,), jnp.int32),
            pltpu.VMEM((D, _TB * 128), jnp.float32),
            pltpu.VMEM((D, _TB * 128), jnp.float32),
            pltpu.VMEM((D, bpw), jnp.float32),
            pltpu.VMEM((D, bpw), jnp.float32),
            pltpu.SemaphoreType.DMA,
            pltpu.SemaphoreType.DMA,
        ],
    )
    def gather(ut_hbm, it_hbm, ui_hbm, ii_hbm, uo_hbm, io_hbm,
               t_u, l_u, t_i, l_i, tiles_u, tiles_i, cols_u, cols_i,
               sem_u, sem_i):
        wid = lax.axis_index("s") * _NC + lax.axis_index("c")
        base = pl.multiple_of(wid * bpw, 128)
        lanes = lax.iota(jnp.int32, _L)

        for i_hbm, t_v, l_v in ((ui_hbm, t_u, l_u), (ii_hbm, t_i, l_i)):
            pltpu.sync_copy(i_hbm.at[pl.ds(base, bpw)], t_v.at[pl.ds(0, bpw)])

            def split_idx(k, _, t_v=t_v, l_v=l_v):
                v = t_v[pl.ds(k * _L, _L)]
                l_v[pl.ds(k * _L, _L)] = lax.bitwise_and(v, 127)
                t_v[pl.ds(k * _L, _L)] = lax.shift_right_logical(v, 7)
                return 0

            lax.fori_loop(0, bpw // _L, split_idx, 0)

        def params(s):
            tab, t_v, tiles, sem = (
                (ut_hbm, t_u, tiles_u, sem_u) if s % 2 == 0
                else (it_hbm, t_i, tiles_i, sem_i))
            return tab, t_v, tiles, sem, (s // 2) * _TB

        def tile_copy(tab, t_v, tiles, sem, row0, j):
            t = t_v[pl.ds(row0 + j, _L)][0]
            return pltpu.make_async_copy(
                tab.at[:, pl.ds(pl.multiple_of(t * 128, 128), 128)],
                tiles.at[:, pl.ds(pl.multiple_of(j * 128, 128), 128)],
                sem)

        def fire(s):
            tab, t_v, tiles, sem, row0 = params(s)
            lax.fori_loop(
                0, _TB,
                lambda j, _: (tile_copy(tab, t_v, tiles, sem, row0, j)
                              .start(), 0)[1],
                0)

        def drain_extract(s):
            tab, t_v, tiles, sem, row0 = params(s)
            l_v = l_u if s % 2 == 0 else l_i
            cols = cols_u if s % 2 == 0 else cols_i
            lax.fori_loop(
                0, _TB,
                lambda j, _: (tile_copy(tab, t_v, tiles, sem, row0, j)
                              .wait(), 0)[1],
                0)
            lane_vec = l_v[pl.ds(row0, _L)]
            src1 = lanes * 128 + lane_vec
            dst1 = lanes + row0
            for d in range(D):
                d_vec = jnp.full((_L,), d, jnp.int32)
                vals = plsc.load_gather(tiles, [d_vec, src1])
                plsc.store_scatter(cols, [d_vec, dst1], vals)

        fire(0)
        for s in range(2 * nb):
            if s + 1 < 2 * nb:
                fire(s + 1)
            drain_extract(s)

        pltpu.sync_copy(cols_u, uo_hbm.at[:, pl.ds(base, bpw)])
        pltpu.sync_copy(cols_i, io_hbm.at[:, pl.ds(base, bpw)])

    return gather(user_table_t, item_table_t, uidx, iidx)


_LOG2E = 1.4426950408889634
_LN2 = 0.6931471805599453


def _make_loss_body(col0):
    def _loss_body(ut_ref, vt_ref, out_ref):
        i = pl.program_id(0)
        tm = ut_ref.shape[1]
        u = ut_ref[...] * _LOG2E
        logits2 = lax.dot_general(u, vt_ref[...],
                                  (((0,), (0,)), ((), ())),
                                  preferred_element_type=jnp.float32)
        sp = lax.log(1.0 + lax.exp2(logits2))
        diag2 = jnp.sum(u * vt_ref[:, pl.ds(col0 + i * tm, tm)])
        part = jnp.sum(sp) - diag2 * _LN2

        @pl.when(i == 0)
        def _init():
            out_ref[0, 0] = 0.0

        out_ref[0, 0] += part

    return _loss_body


def _loss_tc(u_emb_t, v_emb_t, col0):
    D, M = u_emb_t.shape
    B = v_emb_t.shape[1]
    out = pl.pallas_call(
        _make_loss_body(col0),
        grid=(M // _TM,),
        in_specs=[
            pl.BlockSpec((D, _TM), lambda i: (0, i)),
            pl.BlockSpec((D, B), lambda i: (0, 0)),
        ],
        out_specs=pl.BlockSpec(block_shape=(1, 1), index_map=lambda i: (0, 0),
                               memory_space=pltpu.SMEM),
        out_shape=jax.ShapeDtypeStruct((1, 1), jnp.float32),
        compiler_params=pltpu.CompilerParams(
            vmem_limit_bytes=100 * 1024 * 1024),
    )(u_emb_t, v_emb_t)
    return out[0, 0]


def kernel(userid, itemid, user_feature, item_feature, user_table, item_table):
    B = userid.shape[0]
    uidx = userid.reshape(-1)
    iidx = itemid.reshape(-1)
    u_emb_t, i_emb_t = _sc_gather(user_table.T, item_table.T, uidx, iidx)
    total = _loss_tc(u_emb_t, i_emb_t, 0)
    return total / jnp.float32(B)
